# baseline (speedup 1.0000x reference)
import functools

import jax
import jax.numpy as jnp
from jax import lax
from jax.experimental import pallas as pl
from jax.experimental.pallas import tpu as pltpu
from jax.experimental.pallas import tpu_sc as plsc

_NBUF = 2
_LANES = 128


def _sc_mesh():
    return plsc.VectorSubcoreMesh(core_axis_name="c", subcore_axis_name="s")


@functools.lru_cache(maxsize=None)
def _make_transpose(V, D):
    W = 512

    def body(tt_ref, out_ref):
        out_ref[:, :D] = jnp.transpose(tt_ref[...], (1, 0))

    return pl.pallas_call(
        body,
        grid=(pl.cdiv(V, W),),
        in_specs=[pl.BlockSpec((D, W), lambda i: (0, i))],
        out_specs=pl.BlockSpec((W, _LANES), lambda i: (i, 0)),
        out_shape=jax.ShapeDtypeStruct((V, _LANES), jnp.float32),
    )


@functools.lru_cache(maxsize=None)
def _make_gather(V, B):
    info = plsc.get_sparse_core_info()
    NC, NS = info.num_cores, info.num_subcores
    NW = NC * NS
    assert B % NW == 0
    b_per_w = B // NW
    C = 256
    assert b_per_w % (C * _NBUF) == 0
    n_chunks = b_per_w // C
    n_groups = n_chunks // _NBUF

    @functools.partial(
        pl.kernel,
        mesh=_sc_mesh(),
        out_type=jax.ShapeDtypeStruct((B, _LANES), jnp.float32),
        compiler_params=pltpu.CompilerParams(use_tc_tiling_on_sc=True),
        scratch_types=[
            [pltpu.VMEM((C,), jnp.int32)] * _NBUF,
            [pltpu.VMEM((C, _LANES), jnp.float32)] * _NBUF,
            [pltpu.SemaphoreType.DMA] * _NBUF,
            [pltpu.SemaphoreType.DMA] * _NBUF,
        ],
    )
    def k2(idx_hbm, table_hbm, out_hbm, idx_v, rows_v, gsem, ssem):
        wid = lax.axis_index("s") * NC + lax.axis_index("c")
        w_base = wid * b_per_w

        def start_gather(chunk, b):
            pltpu.sync_copy(idx_hbm.at[pl.ds(w_base + chunk * C, C)],
                            idx_v[b])
            pltpu.async_copy(table_hbm.at[idx_v[b]], rows_v[b], gsem[b])

        def wait_gather(b):
            pltpu.make_async_copy(table_hbm.at[idx_v[b]], rows_v[b],
                                  gsem[b]).wait()

        def start_store(chunk, b):
            pltpu.async_copy(rows_v[b],
                             out_hbm.at[pl.ds(w_base + chunk * C, C)], ssem[b])

        def wait_store(chunk, b):
            pltpu.make_async_copy(rows_v[b],
                                  out_hbm.at[pl.ds(w_base + chunk * C, C)],
                                  ssem[b]).wait()

        for b in range(_NBUF):
            start_gather(b, b)

        def body(g, carry):
            for b in range(_NBUF):
                i = g * _NBUF + b
                wait_gather(b)
                start_store(i, b)
                wait_store(i, b)
                start_gather(i + _NBUF, b)
            return carry

        lax.fori_loop(0, n_groups - 1, body, 0)

        for b in range(_NBUF):
            i = (n_groups - 1) * _NBUF + b
            wait_gather(b)
            pltpu.sync_copy(rows_v[b], out_hbm.at[pl.ds(w_base + i * C, C)])

    return k2


def kernel(input_x, table):
    Bt, H = input_x.shape
    V, D = table.shape
    tt = jnp.transpose(table)
    tp = _make_transpose(V, D)(tt)
    idx = input_x.reshape(-1)
    out = _make_gather(V, idx.shape[0])(idx, tp)
    return out[:, :D].reshape(Bt, H, D)


# speedup vs baseline: 1.7096x; 1.7096x over previous
import functools

import jax
import jax.numpy as jnp
from jax import lax
from jax.experimental import pallas as pl
from jax.experimental.pallas import tpu as pltpu
from jax.experimental.pallas import tpu_sc as plsc

_NBUF = 2
_LANES = 128


def _sc_mesh():
    return plsc.VectorSubcoreMesh(core_axis_name="c", subcore_axis_name="s")


@functools.lru_cache(maxsize=None)
def _make_transpose(V, D):
    W = 8192

    def body(tt_ref, out_ref):
        t = jnp.transpose(tt_ref[...], (1, 0))
        out_ref[...] = jnp.concatenate([t, t], axis=1)

    return pl.pallas_call(
        body,
        grid=(pl.cdiv(V, W),),
        in_specs=[pl.BlockSpec((D, W), lambda i: (0, i))],
        out_specs=pl.BlockSpec((W, _LANES), lambda i: (i, 0)),
        out_shape=jax.ShapeDtypeStruct((V, _LANES), jnp.float32),
    )


@functools.lru_cache(maxsize=None)
def _make_gather(V, B):
    info = plsc.get_sparse_core_info()
    NC, NS = info.num_cores, info.num_subcores
    NW = NC * NS
    assert B % NW == 0
    b_per_w = B // NW
    C = 256
    assert b_per_w % (C * _NBUF) == 0
    n_chunks = b_per_w // C
    n_groups = n_chunks // _NBUF

    @functools.partial(
        pl.kernel,
        mesh=_sc_mesh(),
        out_type=jax.ShapeDtypeStruct((B, _LANES), jnp.float32),
        compiler_params=pltpu.CompilerParams(use_tc_tiling_on_sc=True),
        scratch_types=[
            [pltpu.VMEM((C,), jnp.int32)] * _NBUF,
            [pltpu.VMEM((C, _LANES), jnp.float32)] * _NBUF,
            [pltpu.SemaphoreType.DMA] * _NBUF,
            [pltpu.SemaphoreType.DMA] * _NBUF,
        ],
    )
    def k2(idx_hbm, table_hbm, out_hbm, idx_v, rows_v, gsem, ssem):
        wid = lax.axis_index("s") * NC + lax.axis_index("c")
        w_base = wid * b_per_w

        def start_gather(chunk, b):
            pltpu.sync_copy(idx_hbm.at[pl.ds(w_base + chunk * C, C)],
                            idx_v[b])
            pltpu.async_copy(table_hbm.at[idx_v[b]], rows_v[b], gsem[b])

        def wait_gather(b):
            pltpu.make_async_copy(table_hbm.at[idx_v[b]], rows_v[b],
                                  gsem[b]).wait()

        def start_store(chunk, b):
            pltpu.async_copy(rows_v[b],
                             out_hbm.at[pl.ds(w_base + chunk * C, C)], ssem[b])

        def wait_store(chunk, b):
            pltpu.make_async_copy(rows_v[b],
                                  out_hbm.at[pl.ds(w_base + chunk * C, C)],
                                  ssem[b]).wait()

        for b in range(_NBUF):
            start_gather(b, b)

        def body(g, carry):
            for b in range(_NBUF):
                i = g * _NBUF + b
                wait_gather(b)
                start_store(i, b)
                wait_store(i, b)
                start_gather(i + _NBUF, b)
            return carry

        lax.fori_loop(0, n_groups - 1, body, 0)

        for b in range(_NBUF):
            i = (n_groups - 1) * _NBUF + b
            wait_gather(b)
            pltpu.sync_copy(rows_v[b], out_hbm.at[pl.ds(w_base + i * C, C)])

    return k2


def kernel(input_x, table):
    Bt, H = input_x.shape
    V, D = table.shape
    tt = jnp.transpose(table)
    tp = _make_transpose(V, D)(tt)
    idx = input_x.reshape(-1)
    out = _make_gather(V, idx.shape[0])(idx, tp)
    return out[:, :D].reshape(Bt, H, D)


# speedup vs baseline: 1.7878x; 1.0457x over previous
import functools

import jax
import jax.numpy as jnp
from jax import lax
from jax.experimental import pallas as pl
from jax.experimental.pallas import tpu as pltpu
from jax.experimental.pallas import tpu_sc as plsc

_NBUF = 2
_LANES = 128


def _sc_mesh():
    return plsc.VectorSubcoreMesh(core_axis_name="c", subcore_axis_name="s")


@functools.lru_cache(maxsize=None)
def _make_transpose(V, D):
    W = 8192

    def body(tt_ref, out_ref):
        out_ref[:, :D] = jnp.transpose(tt_ref[...], (1, 0))

    return pl.pallas_call(
        body,
        grid=(pl.cdiv(V, W),),
        in_specs=[pl.BlockSpec((D, W), lambda i: (0, i))],
        out_specs=pl.BlockSpec((W, _LANES), lambda i: (i, 0)),
        out_shape=jax.ShapeDtypeStruct((V, _LANES), jnp.float32),
    )


@functools.lru_cache(maxsize=None)
def _make_gather(V, B):
    info = plsc.get_sparse_core_info()
    NC, NS = info.num_cores, info.num_subcores
    NW = NC * NS
    assert B % NW == 0
    b_per_w = B // NW
    C = 256
    assert b_per_w % (C * _NBUF) == 0
    n_chunks = b_per_w // C
    n_groups = n_chunks // _NBUF

    @functools.partial(
        pl.kernel,
        mesh=_sc_mesh(),
        out_type=jax.ShapeDtypeStruct((B, _LANES), jnp.float32),
        compiler_params=pltpu.CompilerParams(use_tc_tiling_on_sc=True),
        scratch_types=[
            [pltpu.VMEM((C,), jnp.int32)] * _NBUF,
            [pltpu.VMEM((C, _LANES), jnp.float32)] * _NBUF,
            [pltpu.SemaphoreType.DMA] * _NBUF,
            [pltpu.SemaphoreType.DMA] * _NBUF,
        ],
    )
    def k2(idx_hbm, table_hbm, out_hbm, idx_v, rows_v, gsem, ssem):
        wid = lax.axis_index("s") * NC + lax.axis_index("c")
        w_base = wid * b_per_w

        def start_gather(chunk, b):
            pltpu.sync_copy(idx_hbm.at[pl.ds(w_base + chunk * C, C)],
                            idx_v[b])
            pltpu.async_copy(table_hbm.at[idx_v[b]], rows_v[b], gsem[b])

        def wait_gather(b):
            pltpu.make_async_copy(table_hbm.at[idx_v[b]], rows_v[b],
                                  gsem[b]).wait()

        def start_store(chunk, b):
            pltpu.async_copy(rows_v[b],
                             out_hbm.at[pl.ds(w_base + chunk * C, C)], ssem[b])

        def wait_store(chunk, b):
            pltpu.make_async_copy(rows_v[b],
                                  out_hbm.at[pl.ds(w_base + chunk * C, C)],
                                  ssem[b]).wait()

        for b in range(_NBUF):
            start_gather(b, b)

        def body(g, carry):
            for b in range(_NBUF):
                i = g * _NBUF + b
                wait_gather(b)
                start_store(i, b)
                wait_store(i, b)
                start_gather(i + _NBUF, b)
            return carry

        lax.fori_loop(0, n_groups - 1, body, 0)

        for b in range(_NBUF):
            i = (n_groups - 1) * _NBUF + b
            wait_gather(b)
            pltpu.sync_copy(rows_v[b], out_hbm.at[pl.ds(w_base + i * C, C)])

    return k2


def kernel(input_x, table):
    Bt, H = input_x.shape
    V, D = table.shape
    tt = jnp.transpose(table)
    tp = _make_transpose(V, D)(tt)
    idx = input_x.reshape(-1)
    out = _make_gather(V, idx.shape[0])(idx, tp)
    return out[:, :D].reshape(Bt, H, D)


# speedup vs baseline: 2.2091x; 1.2356x over previous
import functools

import jax
import jax.numpy as jnp
from jax import lax
from jax.experimental import pallas as pl
from jax.experimental.pallas import tpu as pltpu
from jax.experimental.pallas import tpu_sc as plsc

_NBUF = 2
_LANES = 128


def _sc_mesh():
    return plsc.VectorSubcoreMesh(core_axis_name="c", subcore_axis_name="s")


@functools.lru_cache(maxsize=None)
def _make_transpose(V, D):
    W = 8192

    def body(tt_ref, out_ref):
        out_ref[:, :D] = jnp.transpose(tt_ref[...], (1, 0))

    return pl.pallas_call(
        body,
        grid=(pl.cdiv(V, W),),
        in_specs=[pl.BlockSpec((D, W), lambda i: (0, i))],
        out_specs=pl.BlockSpec((W, _LANES), lambda i: (i, 0)),
        out_shape=jax.ShapeDtypeStruct((V, _LANES), jnp.float32),
    )


@functools.lru_cache(maxsize=None)
def _make_untile(Bt, H, D, W):
    def body(g_ref, out_ref):
        x = g_ref[...].reshape(W, H, _LANES)
        out_ref[...] = jnp.transpose(x[:, :, :D], (1, 2, 0))

    return pl.pallas_call(
        body,
        grid=(Bt // W,),
        in_specs=[pl.BlockSpec((W * H, _LANES), lambda i: (i, 0))],
        out_specs=pl.BlockSpec((H, D, W), lambda i: (0, 0, i)),
        out_shape=jax.ShapeDtypeStruct((H, D, Bt), jnp.float32),
    )


@functools.lru_cache(maxsize=None)
def _make_gather(V, B):
    info = plsc.get_sparse_core_info()
    NC, NS = info.num_cores, info.num_subcores
    NW = NC * NS
    assert B % NW == 0
    b_per_w = B // NW
    C = 256
    assert b_per_w % (C * _NBUF) == 0
    n_chunks = b_per_w // C
    n_groups = n_chunks // _NBUF

    @functools.partial(
        pl.kernel,
        mesh=_sc_mesh(),
        out_type=jax.ShapeDtypeStruct((B, _LANES), jnp.float32),
        compiler_params=pltpu.CompilerParams(use_tc_tiling_on_sc=True),
        scratch_types=[
            [pltpu.VMEM((C,), jnp.int32)] * _NBUF,
            [pltpu.VMEM((C, _LANES), jnp.float32)] * _NBUF,
            [pltpu.SemaphoreType.DMA] * _NBUF,
            [pltpu.SemaphoreType.DMA] * _NBUF,
        ],
    )
    def k2(idx_hbm, table_hbm, out_hbm, idx_v, rows_v, gsem, ssem):
        wid = lax.axis_index("s") * NC + lax.axis_index("c")
        w_base = wid * b_per_w

        def start_gather(chunk, b):
            pltpu.sync_copy(idx_hbm.at[pl.ds(w_base + chunk * C, C)],
                            idx_v[b])
            pltpu.async_copy(table_hbm.at[idx_v[b]], rows_v[b], gsem[b])

        def wait_gather(b):
            pltpu.make_async_copy(table_hbm.at[idx_v[b]], rows_v[b],
                                  gsem[b]).wait()

        def start_store(chunk, b):
            pltpu.async_copy(rows_v[b],
                             out_hbm.at[pl.ds(w_base + chunk * C, C)], ssem[b])

        def wait_store(chunk, b):
            pltpu.make_async_copy(rows_v[b],
                                  out_hbm.at[pl.ds(w_base + chunk * C, C)],
                                  ssem[b]).wait()

        for b in range(_NBUF):
            start_gather(b, b)

        def body(g, carry):
            for b in range(_NBUF):
                i = g * _NBUF + b
                wait_gather(b)
                start_store(i, b)
                wait_store(i, b)
                start_gather(i + _NBUF, b)
            return carry

        lax.fori_loop(0, n_groups - 1, body, 0)

        for b in range(_NBUF):
            i = (n_groups - 1) * _NBUF + b
            wait_gather(b)
            pltpu.sync_copy(rows_v[b], out_hbm.at[pl.ds(w_base + i * C, C)])

    return k2


def kernel(input_x, table):
    Bt, H = input_x.shape
    V, D = table.shape
    tt = jnp.transpose(table)
    tp = _make_transpose(V, D)(tt)
    idx = input_x.reshape(-1)
    out = _make_gather(V, idx.shape[0])(idx, tp)
    out_p = _make_untile(Bt, H, D, 256)(out)
    return jnp.transpose(out_p, (2, 0, 1))


# speedup vs baseline: 2.2380x; 1.0131x over previous
import functools

import jax
import jax.numpy as jnp
from jax import lax
from jax.experimental import pallas as pl
from jax.experimental.pallas import tpu as pltpu
from jax.experimental.pallas import tpu_sc as plsc

_NBUF = 2
_LANES = 128


def _sc_mesh():
    return plsc.VectorSubcoreMesh(core_axis_name="c", subcore_axis_name="s")


@functools.lru_cache(maxsize=None)
def _make_transpose(V, D):
    W = 16384

    def body(tt_ref, out_ref):
        out_ref[:, :D] = jnp.transpose(tt_ref[...], (1, 0))

    return pl.pallas_call(
        body,
        grid=(pl.cdiv(V, W),),
        in_specs=[pl.BlockSpec((D, W), lambda i: (0, i))],
        out_specs=pl.BlockSpec((W, _LANES), lambda i: (i, 0)),
        out_shape=jax.ShapeDtypeStruct((V, _LANES), jnp.float32),
    )


@functools.lru_cache(maxsize=None)
def _make_untile(Bt, H, D, W):
    def body(g_ref, out_ref):
        x = g_ref[...].reshape(W, H, _LANES)
        out_ref[...] = jnp.transpose(x[:, :, :D], (1, 2, 0))

    return pl.pallas_call(
        body,
        grid=(Bt // W,),
        in_specs=[pl.BlockSpec((W * H, _LANES), lambda i: (i, 0))],
        out_specs=pl.BlockSpec((H, D, W), lambda i: (0, 0, i)),
        out_shape=jax.ShapeDtypeStruct((H, D, Bt), jnp.float32),
    )


@functools.lru_cache(maxsize=None)
def _make_gather(V, B):
    info = plsc.get_sparse_core_info()
    NC, NS = info.num_cores, info.num_subcores
    NW = NC * NS
    assert B % NW == 0
    b_per_w = B // NW
    C = 256
    assert b_per_w % (C * _NBUF) == 0
    n_chunks = b_per_w // C
    n_groups = n_chunks // _NBUF

    @functools.partial(
        pl.kernel,
        mesh=_sc_mesh(),
        out_type=jax.ShapeDtypeStruct((B, _LANES), jnp.float32),
        compiler_params=pltpu.CompilerParams(use_tc_tiling_on_sc=True),
        scratch_types=[
            [pltpu.VMEM((C,), jnp.int32)] * _NBUF,
            [pltpu.VMEM((C, _LANES), jnp.float32)] * _NBUF,
            [pltpu.SemaphoreType.DMA] * _NBUF,
            [pltpu.SemaphoreType.DMA] * _NBUF,
        ],
    )
    def k2(idx_hbm, table_hbm, out_hbm, idx_v, rows_v, gsem, ssem):
        wid = lax.axis_index("s") * NC + lax.axis_index("c")
        w_base = wid * b_per_w

        def start_gather(chunk, b):
            pltpu.sync_copy(idx_hbm.at[pl.ds(w_base + chunk * C, C)],
                            idx_v[b])
            pltpu.async_copy(table_hbm.at[idx_v[b]], rows_v[b], gsem[b])

        def wait_gather(b):
            pltpu.make_async_copy(table_hbm.at[idx_v[b]], rows_v[b],
                                  gsem[b]).wait()

        def start_store(chunk, b):
            pltpu.async_copy(rows_v[b],
                             out_hbm.at[pl.ds(w_base + chunk * C, C)], ssem[b])

        def wait_store(chunk, b):
            pltpu.make_async_copy(rows_v[b],
                                  out_hbm.at[pl.ds(w_base + chunk * C, C)],
                                  ssem[b]).wait()

        for b in range(_NBUF):
            start_gather(b, b)

        def body(g, carry):
            for b in range(_NBUF):
                i = g * _NBUF + b
                wait_gather(b)
                start_store(i, b)
                wait_store(i, b)
                start_gather(i + _NBUF, b)
            return carry

        lax.fori_loop(0, n_groups - 1, body, 0)

        for b in range(_NBUF):
            i = (n_groups - 1) * _NBUF + b
            wait_gather(b)
            pltpu.sync_copy(rows_v[b], out_hbm.at[pl.ds(w_base + i * C, C)])

    return k2


def kernel(input_x, table):
    Bt, H = input_x.shape
    V, D = table.shape
    tt = jnp.transpose(table)
    tp = _make_transpose(V, D)(tt)
    idx = input_x.reshape(-1)
    out = _make_gather(V, idx.shape[0])(idx, tp)
    out_p = _make_untile(Bt, H, D, 512)(out)
    return jnp.transpose(out_p, (2, 0, 1))


# speedup vs baseline: 2.4810x; 1.1086x over previous
import functools

import jax
import jax.numpy as jnp
from jax import lax
from jax.experimental import pallas as pl
from jax.experimental.pallas import tpu as pltpu
from jax.experimental.pallas import tpu_sc as plsc

_NBUF = 2
_LANES = 128


def _sc_mesh():
    return plsc.VectorSubcoreMesh(core_axis_name="c", subcore_axis_name="s")


@functools.lru_cache(maxsize=None)
def _make_transpose(V, D):
    W = 16384

    def body(tt_ref, out_ref):
        out_ref[:, :D] = jnp.transpose(tt_ref[...], (1, 0))

    return pl.pallas_call(
        body,
        grid=(pl.cdiv(V, W),),
        in_specs=[pl.BlockSpec((D, W), lambda i: (0, i))],
        out_specs=pl.BlockSpec((W, _LANES), lambda i: (i, 0)),
        out_shape=jax.ShapeDtypeStruct((V, _LANES), jnp.float32),
    )


@functools.lru_cache(maxsize=None)
def _make_untile(Bt, H, D, W, part, nparts, aliased):
    bp = Bt // nparts
    nblk = bp // W
    off = part * nblk

    def body(*refs):
        g_ref, out_ref = refs[-2], refs[-1]
        x = g_ref[...].reshape(W, H, _LANES)
        out_ref[...] = jnp.transpose(x[:, :, :D], (1, 2, 0))

    in_specs = [pl.BlockSpec((W * H, _LANES), lambda i: (i, 0))]
    io_aliases = {}
    if aliased:
        in_specs = [pl.BlockSpec(memory_space=pltpu.MemorySpace.HBM)] + in_specs
        io_aliases = {0: 0}

    return pl.pallas_call(
        body,
        grid=(nblk,),
        in_specs=in_specs,
        out_specs=pl.BlockSpec((H, D, W), lambda i: (0, 0, off + i)),
        out_shape=jax.ShapeDtypeStruct((H, D, Bt), jnp.float32),
        input_output_aliases=io_aliases,
    )


@functools.lru_cache(maxsize=None)
def _make_gather(V, B):
    info = plsc.get_sparse_core_info()
    NC, NS = info.num_cores, info.num_subcores
    NW = NC * NS
    assert B % NW == 0
    b_per_w = B // NW
    C = 256
    assert b_per_w % (C * _NBUF) == 0
    n_chunks = b_per_w // C
    n_groups = n_chunks // _NBUF

    @functools.partial(
        pl.kernel,
        mesh=_sc_mesh(),
        out_type=jax.ShapeDtypeStruct((B, _LANES), jnp.float32),
        compiler_params=pltpu.CompilerParams(use_tc_tiling_on_sc=True),
        scratch_types=[
            [pltpu.VMEM((C,), jnp.int32)] * _NBUF,
            [pltpu.VMEM((C, _LANES), jnp.float32)] * _NBUF,
            [pltpu.SemaphoreType.DMA] * _NBUF,
            [pltpu.SemaphoreType.DMA] * _NBUF,
        ],
    )
    def k2(idx_hbm, table_hbm, out_hbm, idx_v, rows_v, gsem, ssem):
        wid = lax.axis_index("s") * NC + lax.axis_index("c")
        w_base = wid * b_per_w

        def start_gather(chunk, b):
            pltpu.sync_copy(idx_hbm.at[pl.ds(w_base + chunk * C, C)],
                            idx_v[b])
            pltpu.async_copy(table_hbm.at[idx_v[b]], rows_v[b], gsem[b])

        def wait_gather(b):
            pltpu.make_async_copy(table_hbm.at[idx_v[b]], rows_v[b],
                                  gsem[b]).wait()

        def start_store(chunk, b):
            pltpu.async_copy(rows_v[b],
                             out_hbm.at[pl.ds(w_base + chunk * C, C)], ssem[b])

        def wait_store(chunk, b):
            pltpu.make_async_copy(rows_v[b],
                                  out_hbm.at[pl.ds(w_base + chunk * C, C)],
                                  ssem[b]).wait()

        for b in range(_NBUF):
            start_gather(b, b)

        def body(g, carry):
            for b in range(_NBUF):
                i = g * _NBUF + b
                wait_gather(b)
                start_store(i, b)
                wait_store(i, b)
                start_gather(i + _NBUF, b)
            return carry

        lax.fori_loop(0, n_groups - 1, body, 0)

        for b in range(_NBUF):
            i = (n_groups - 1) * _NBUF + b
            wait_gather(b)
            pltpu.sync_copy(rows_v[b], out_hbm.at[pl.ds(w_base + i * C, C)])

    return k2


def kernel(input_x, table):
    Bt, H = input_x.shape
    V, D = table.shape
    tt = jnp.transpose(table)
    tp = _make_transpose(V, D)(tt)
    idx = input_x.reshape(-1)
    NP = 2
    Bh = idx.shape[0] // NP
    parts = [_make_gather(V, Bh)(idx[p * Bh:(p + 1) * Bh], tp)
             for p in range(NP)]
    out_p = _make_untile(Bt, H, D, 512, 0, NP, False)(parts[0])
    for p in range(1, NP):
        out_p = _make_untile(Bt, H, D, 512, p, NP, True)(out_p, parts[p])
    return jnp.transpose(out_p, (2, 0, 1))


# speedup vs baseline: 2.5746x; 1.0377x over previous
import functools

import jax
import jax.numpy as jnp
from jax import lax
from jax.experimental import pallas as pl
from jax.experimental.pallas import tpu as pltpu
from jax.experimental.pallas import tpu_sc as plsc

_NBUF = 2
_LANES = 128


def _sc_mesh():
    return plsc.VectorSubcoreMesh(core_axis_name="c", subcore_axis_name="s")


@functools.lru_cache(maxsize=None)
def _make_transpose(V, D):
    W = 16384

    def body(tt_ref, out_ref):
        out_ref[:, :D] = jnp.transpose(tt_ref[...], (1, 0))

    return pl.pallas_call(
        body,
        grid=(pl.cdiv(V, W),),
        in_specs=[pl.BlockSpec((D, W), lambda i: (0, i))],
        out_specs=pl.BlockSpec((W, _LANES), lambda i: (i, 0)),
        out_shape=jax.ShapeDtypeStruct((V, _LANES), jnp.float32),
    )


@functools.lru_cache(maxsize=None)
def _make_untile(Bt, H, D, W, part, nparts, aliased):
    bp = Bt // nparts
    nblk = bp // W
    off = part * nblk

    def body(*refs):
        g_ref, out_ref = refs[-2], refs[-1]
        x = g_ref[...].reshape(W, H, _LANES)
        out_ref[...] = jnp.transpose(x[:, :, :D], (1, 2, 0))

    in_specs = [pl.BlockSpec((W * H, _LANES), lambda i: (i, 0))]
    io_aliases = {}
    if aliased:
        in_specs = [pl.BlockSpec(memory_space=pltpu.MemorySpace.HBM)] + in_specs
        io_aliases = {0: 0}

    return pl.pallas_call(
        body,
        grid=(nblk,),
        in_specs=in_specs,
        out_specs=pl.BlockSpec((H, D, W), lambda i: (0, 0, off + i)),
        out_shape=jax.ShapeDtypeStruct((H, D, Bt), jnp.float32),
        input_output_aliases=io_aliases,
    )


@functools.lru_cache(maxsize=None)
def _make_gather(V, B):
    info = plsc.get_sparse_core_info()
    NC, NS = info.num_cores, info.num_subcores
    NW = NC * NS
    assert B % NW == 0
    b_per_w = B // NW
    C = 128
    assert b_per_w % (C * _NBUF) == 0
    n_chunks = b_per_w // C
    n_groups = n_chunks // _NBUF

    @functools.partial(
        pl.kernel,
        mesh=_sc_mesh(),
        out_type=jax.ShapeDtypeStruct((B, _LANES), jnp.float32),
        compiler_params=pltpu.CompilerParams(use_tc_tiling_on_sc=True),
        scratch_types=[
            [pltpu.VMEM((C,), jnp.int32)] * _NBUF,
            [pltpu.VMEM((C, _LANES), jnp.float32)] * _NBUF,
            [pltpu.SemaphoreType.DMA] * _NBUF,
            [pltpu.SemaphoreType.DMA] * _NBUF,
        ],
    )
    def k2(idx_hbm, table_hbm, out_hbm, idx_v, rows_v, gsem, ssem):
        wid = lax.axis_index("s") * NC + lax.axis_index("c")
        w_base = wid * b_per_w

        def start_gather(chunk, b):
            pltpu.sync_copy(idx_hbm.at[pl.ds(w_base + chunk * C, C)],
                            idx_v[b])
            pltpu.async_copy(table_hbm.at[idx_v[b]], rows_v[b], gsem[b])

        def wait_gather(b):
            pltpu.make_async_copy(table_hbm.at[idx_v[b]], rows_v[b],
                                  gsem[b]).wait()

        def start_store(chunk, b):
            pltpu.async_copy(rows_v[b],
                             out_hbm.at[pl.ds(w_base + chunk * C, C)], ssem[b])

        def wait_store(chunk, b):
            pltpu.make_async_copy(rows_v[b],
                                  out_hbm.at[pl.ds(w_base + chunk * C, C)],
                                  ssem[b]).wait()

        for b in range(_NBUF):
            start_gather(b, b)

        def body(g, carry):
            for b in range(_NBUF):
                i = g * _NBUF + b
                wait_gather(b)
                start_store(i, b)
                wait_store(i, b)
                start_gather(i + _NBUF, b)
            return carry

        lax.fori_loop(0, n_groups - 1, body, 0)

        for b in range(_NBUF):
            i = (n_groups - 1) * _NBUF + b
            wait_gather(b)
            pltpu.sync_copy(rows_v[b], out_hbm.at[pl.ds(w_base + i * C, C)])

    return k2


def kernel(input_x, table):
    Bt, H = input_x.shape
    V, D = table.shape
    tt = jnp.transpose(table)
    tp = _make_transpose(V, D)(tt)
    idx = input_x.reshape(-1)
    NP = 4
    Bh = idx.shape[0] // NP
    parts = [_make_gather(V, Bh)(idx[p * Bh:(p + 1) * Bh], tp)
             for p in range(NP)]
    out_p = _make_untile(Bt, H, D, 512, 0, NP, False)(parts[0])
    for p in range(1, NP):
        out_p = _make_untile(Bt, H, D, 512, p, NP, True)(out_p, parts[p])
    return jnp.transpose(out_p, (2, 0, 1))


# speedup vs baseline: 2.5908x; 1.0063x over previous
import functools

import jax
import jax.numpy as jnp
from jax import lax
from jax.experimental import pallas as pl
from jax.experimental.pallas import tpu as pltpu
from jax.experimental.pallas import tpu_sc as plsc

_NBUF = 2
_LANES = 128


def _sc_mesh():
    return plsc.VectorSubcoreMesh(core_axis_name="c", subcore_axis_name="s")


@functools.lru_cache(maxsize=None)
def _make_transpose(V, D):
    W = 32768

    def body(tt_ref, out_ref):
        out_ref[:, :D] = jnp.transpose(tt_ref[...], (1, 0))

    return pl.pallas_call(
        body,
        grid=(pl.cdiv(V, W),),
        in_specs=[pl.BlockSpec((D, W), lambda i: (0, i))],
        out_specs=pl.BlockSpec((W, _LANES), lambda i: (i, 0)),
        out_shape=jax.ShapeDtypeStruct((V, _LANES), jnp.float32),
    )


@functools.lru_cache(maxsize=None)
def _make_untile(Bt, H, D, W, part, nparts, aliased):
    bp = Bt // nparts
    nblk = bp // W
    off = part * nblk

    def body(*refs):
        g_ref, out_ref = refs[-2], refs[-1]
        x = g_ref[...].reshape(W, H, _LANES)
        out_ref[...] = jnp.transpose(x[:, :, :D], (1, 2, 0))

    in_specs = [pl.BlockSpec((W * H, _LANES), lambda i: (i, 0))]
    io_aliases = {}
    if aliased:
        in_specs = [pl.BlockSpec(memory_space=pltpu.MemorySpace.HBM)] + in_specs
        io_aliases = {0: 0}

    return pl.pallas_call(
        body,
        grid=(nblk,),
        in_specs=in_specs,
        out_specs=pl.BlockSpec((H, D, W), lambda i: (0, 0, off + i)),
        out_shape=jax.ShapeDtypeStruct((H, D, Bt), jnp.float32),
        input_output_aliases=io_aliases,
    )


@functools.lru_cache(maxsize=None)
def _make_gather(V, B):
    info = plsc.get_sparse_core_info()
    NC, NS = info.num_cores, info.num_subcores
    NW = NC * NS
    assert B % NW == 0
    b_per_w = B // NW
    C = 128
    assert b_per_w % (C * _NBUF) == 0
    n_chunks = b_per_w // C
    n_groups = n_chunks // _NBUF

    @functools.partial(
        pl.kernel,
        mesh=_sc_mesh(),
        out_type=jax.ShapeDtypeStruct((B, _LANES), jnp.float32),
        compiler_params=pltpu.CompilerParams(use_tc_tiling_on_sc=True),
        scratch_types=[
            [pltpu.VMEM((C,), jnp.int32)] * _NBUF,
            [pltpu.VMEM((C, _LANES), jnp.float32)] * _NBUF,
            [pltpu.SemaphoreType.DMA] * _NBUF,
            [pltpu.SemaphoreType.DMA] * _NBUF,
        ],
    )
    def k2(idx_hbm, table_hbm, out_hbm, idx_v, rows_v, gsem, ssem):
        wid = lax.axis_index("s") * NC + lax.axis_index("c")
        w_base = wid * b_per_w

        def start_gather(chunk, b):
            pltpu.sync_copy(idx_hbm.at[pl.ds(w_base + chunk * C, C)],
                            idx_v[b])
            pltpu.async_copy(table_hbm.at[idx_v[b]], rows_v[b], gsem[b])

        def wait_gather(b):
            pltpu.make_async_copy(table_hbm.at[idx_v[b]], rows_v[b],
                                  gsem[b]).wait()

        def start_store(chunk, b):
            pltpu.async_copy(rows_v[b],
                             out_hbm.at[pl.ds(w_base + chunk * C, C)], ssem[b])

        def wait_store(chunk, b):
            pltpu.make_async_copy(rows_v[b],
                                  out_hbm.at[pl.ds(w_base + chunk * C, C)],
                                  ssem[b]).wait()

        for b in range(_NBUF):
            start_gather(b, b)

        def body(g, carry):
            for b in range(_NBUF):
                i = g * _NBUF + b
                wait_gather(b)
                start_store(i, b)
                wait_store(i, b)
                start_gather(i + _NBUF, b)
            return carry

        lax.fori_loop(0, n_groups - 1, body, 0)

        for b in range(_NBUF):
            i = (n_groups - 1) * _NBUF + b
            wait_gather(b)
            pltpu.sync_copy(rows_v[b], out_hbm.at[pl.ds(w_base + i * C, C)])

    return k2


def kernel(input_x, table):
    Bt, H = input_x.shape
    V, D = table.shape
    tt = jnp.transpose(table)
    tp = _make_transpose(V, D)(tt)
    idx = input_x.reshape(-1)
    NP = 4
    Bh = idx.shape[0] // NP
    parts = [_make_gather(V, Bh)(idx[p * Bh:(p + 1) * Bh], tp)
             for p in range(NP)]
    out_p = _make_untile(Bt, H, D, 512, 0, NP, False)(parts[0])
    for p in range(1, NP):
        out_p = _make_untile(Bt, H, D, 512, p, NP, True)(out_p, parts[p])
    return jnp.transpose(out_p, (2, 0, 1))
